# speedup vs baseline: 1.2326x; 1.2326x over previous
import functools

import jax
import jax.numpy as jnp
from jax import lax
from jax.experimental import pallas as pl
from jax.experimental.pallas import tpu as pltpu
from jax.experimental.pallas import tpu_sc as plsc

_E = 8
_DIN = 1024
_DOUT = 1024
_BS = 256
_NW = 32
_CH = 64


def _make_row_gather(n_rows, n_cols, chunk):
    per_w = n_rows // _NW
    n_ch = per_w // chunk
    assert per_w % chunk == 0 and n_rows % _NW == 0 and (per_w % 8 == 0)
    mesh = plsc.VectorSubcoreMesh(core_axis_name="c", subcore_axis_name="s")

    @functools.partial(
        pl.kernel,
        mesh=mesh,
        out_type=jax.ShapeDtypeStruct((n_rows, n_cols), jnp.float32),
        scratch_types=[
            pltpu.VMEM((chunk,), jnp.int32),
            pltpu.VMEM((chunk, n_cols), jnp.float32),
            pltpu.SemaphoreType.DMA,
        ],
    )
    def gather(table_hbm, idx_hbm, out_hbm, idx_v, rows_v, sem):
        wid = lax.axis_index("s") * 2 + lax.axis_index("c")
        base = wid * per_w

        def body(c, carry):
            off = base + c * chunk
            pltpu.sync_copy(idx_hbm.at[pl.ds(off, chunk)], idx_v)
            pltpu.async_copy(table_hbm.at[idx_v], rows_v, sem).wait()
            pltpu.sync_copy(rows_v, out_hbm.at[pl.ds(off, chunk)])
            return carry

        lax.fori_loop(0, n_ch, body, 0)

    return gather


def _mm_body(be_ref, x_ref, w_ref, b_ref, o_ref):
    w = w_ref[0]
    y = lax.dot_general(
        x_ref[...], w, (((1,), (1,)), ((), ())),
        preferred_element_type=jnp.float32)
    o_ref[...] = y + b_ref[0]


def _grouped_matmul(x_sorted, w_t, bias_t, block_expert):
    n_pad = x_sorted.shape[0]
    nb = n_pad // _BS
    grid_spec = pltpu.PrefetchScalarGridSpec(
        num_scalar_prefetch=1,
        grid=(nb,),
        in_specs=[
            pl.BlockSpec((_BS, _DIN), lambda i, be: (i, 0)),
            pl.BlockSpec((1, _DOUT, _DIN), lambda i, be: (be[i], 0, 0)),
            pl.BlockSpec((1, 1, _DOUT), lambda i, be: (be[i], 0, 0)),
        ],
        out_specs=pl.BlockSpec((_BS, _DOUT), lambda i, be: (i, 0)),
    )
    return pl.pallas_call(
        _mm_body,
        grid_spec=grid_spec,
        out_shape=jax.ShapeDtypeStruct((n_pad, _DOUT), jnp.float32),
    )(block_expert, x_sorted, w_t, bias_t)


def kernel(input, topk_indices, expert_weights, expert_biases):
    b, s, k, din = input.shape
    n = b * s * k
    n_pad = n + _E * _BS
    nb = n_pad // _BS

    x = input.reshape(n, din)
    idx = topk_indices.reshape(n).astype(jnp.int32)

    onehot = (idx[:, None] == jnp.arange(_E, dtype=jnp.int32)[None, :])
    csum = jnp.cumsum(onehot.astype(jnp.int32), axis=0)
    counts = csum[-1]
    rank = jnp.take_along_axis(csum, idx[:, None], axis=1)[:, 0] - 1
    padded = ((counts + _BS - 1) // _BS) * _BS
    ends = jnp.cumsum(padded)
    starts = ends - padded
    dest = starts[idx] + rank
    perm = jnp.zeros((n_pad,), jnp.int32).at[dest].set(
        jnp.arange(n, dtype=jnp.int32))
    block_expert = jnp.minimum(
        jnp.searchsorted(ends, jnp.arange(nb, dtype=jnp.int32) * _BS,
                         side="right"),
        _E - 1).astype(jnp.int32)

    x_sorted = _make_row_gather(n_pad, din, _CH)(x, perm)

    w_t = expert_weights.transpose(1, 0, 2)
    bias_t = expert_biases[:, :, 0].T.reshape(_E, 1, _DOUT)
    y_sorted = _grouped_matmul(x_sorted, w_t, bias_t, block_expert)

    out = _make_row_gather(n, _DOUT, _CH)(y_sorted, dest)
    return out.reshape(b, s, k, _DOUT)
